# baseline (speedup 1.0000x reference)
import functools

import jax
import jax.numpy as jnp
from jax import lax
from jax.experimental import pallas as pl
from jax.experimental.pallas import tpu as pltpu
from jax.experimental.pallas import tpu_sc as plsc

NC = 2
NS = 16
NW = NC * NS
L = 16


def _tables_body(z_ref, w1_ref, b1_ref, xn_ref, p_ref, q_ref):
    c = jnp.dot(xn_ref[...], w1_ref[256:384, :],
                preferred_element_type=jnp.float32) + b1_ref[...]
    c = c * 0.5
    p_ref[...] = jnp.dot(z_ref[...], w1_ref[0:128, :],
                         preferred_element_type=jnp.float32) + c
    q_ref[...] = jnp.dot(z_ref[...], w1_ref[128:256, :],
                         preferred_element_type=jnp.float32) + c


def _build_tables(z, W1, b1, xn):
    n, _ = z.shape
    h = W1.shape[1]
    out = jax.ShapeDtypeStruct((n, h), jnp.float32)
    return pl.pallas_call(_tables_body, out_shape=(out, out))(z, W1, b1, xn)


def _sc_edge_body(p_hbm, q_hbm, rows_hbm, cols_hbm, w2_hbm, out_hbm,
                  ridx0, ridx1, ridx2, ridx3, cidx0, cidx1, cidx2, cidx3,
                  qrows0, qrows1, prows0, prows1, out0, out1, w2_v,
                  ir0, ir1, ir2, ir3, ic0, ic1, ic2, ic3,
                  qs0, qs1, ps0, ps1, os0, os1,
                  *, edges_per_w, blk, hidden):
    wid = lax.axis_index("c") * NS + lax.axis_index("s")
    base = wid * edges_per_w
    nstep = edges_per_w // blk
    nchunk = hidden // L
    pltpu.sync_copy(w2_hbm, w2_v)

    ridx = (ridx0, ridx1, ridx2, ridx3)
    cidx = (cidx0, cidx1, cidx2, cidx3)
    irs = (ir0, ir1, ir2, ir3)
    ics = (ic0, ic1, ic2, ic3)
    qrows = (qrows0, qrows1)
    prows = (prows0, prows1)
    outs = (out0, out1)
    qsems = (qs0, qs1)
    psems = (ps0, ps1)
    osems = (os0, os1)

    def start_idx(j):
        s = j % 4
        off = base + j * blk
        dr = pltpu.make_async_copy(rows_hbm.at[pl.ds(off, blk)], ridx[s],
                                   irs[s])
        dc = pltpu.make_async_copy(cols_hbm.at[pl.ds(off, blk)], cidx[s],
                                   ics[s])
        dr.start()
        dc.start()
        return dr, dc

    def start_gathers(j):
        s, b = j % 4, j % 2
        dq = pltpu.make_async_copy(q_hbm.at[ridx[s]], qrows[b], qsems[b])
        dp = pltpu.make_async_copy(p_hbm.at[cidx[s]], prows[b], psems[b])
        dq.start()
        dp.start()
        return dq, dp

    def start_out(j):
        b = j % 2
        d = pltpu.make_async_copy(
            outs[b], out_hbm.at[pl.ds(base + j * blk, blk), :], osems[b])
        d.start()
        return d

    def compute(b):
        pr, qr, ob = prows[b], qrows[b], outs[b]
        b2v = w2_v[pl.ds(hidden, L)]

        @plsc.parallel_loop(0, blk, unroll=8)
        def _edge(e):
            s = b2v
            for ch in range(nchunk):
                sl = pl.ds(ch * L, L)
                s = s + jnp.maximum(pr[e, sl] + qr[e, sl], 0.0) * w2_v[sl]
            ob[e, :] = s

    i0 = start_idx(0)
    i1 = start_idx(1) if nstep > 1 else None
    i0[0].wait()
    i0[1].wait()
    g = start_gathers(0)
    g[0].wait()
    g[1].wait()
    out_pending = {}
    for j in range(nstep):
        g_next = None
        if j + 1 < nstep:
            if j == 0 and i1 is not None:
                i1[0].wait()
                i1[1].wait()
            g_next = start_gathers(j + 1)
        i_next = start_idx(j + 2) if j + 2 < nstep else None
        if j >= 2:
            out_pending.pop(j - 2).wait()
        compute(j % 2)
        out_pending[j] = start_out(j)
        if g_next is not None:
            g_next[0].wait()
            g_next[1].wait()
        if i_next is not None:
            i_next[0].wait()
            i_next[1].wait()
    for d in out_pending.values():
        d.wait()


def _sc_edge(P, Q, rows, cols, w2):
    n, hidden = P.shape
    e = rows.shape[0]
    edges_per_w = e // NW
    blk = 400
    mesh = plsc.VectorSubcoreMesh(core_axis_name="c", subcore_axis_name="s")
    body = functools.partial(_sc_edge_body, edges_per_w=edges_per_w, blk=blk,
                             hidden=hidden)
    idx_t = pltpu.VMEM((blk,), jnp.int32)
    tab_t = pltpu.VMEM((blk, hidden), jnp.float32)
    out_t = pltpu.VMEM((blk, L), jnp.float32)
    run = pl.kernel(
        body,
        out_type=jax.ShapeDtypeStruct((e, L), jnp.float32),
        mesh=mesh,
        compiler_params=pltpu.CompilerParams(use_tc_tiling_on_sc=False),
        scratch_types=(
            [idx_t] * 8 + [tab_t] * 4 + [out_t] * 2
            + [pltpu.VMEM((hidden + L,), jnp.float32)]
            + [pltpu.SemaphoreType.DMA] * 14
        ),
    )
    return run(P, Q, rows, cols, w2)


def _reduce_body(x_ref, o_ref):
    sel = (lax.broadcasted_iota(jnp.int32, (128, 8), 0) // L
           == lax.broadcasted_iota(jnp.int32, (128, 8), 1))
    o_ref[...] = jnp.dot(x_ref[...], sel.astype(jnp.float32),
                         preferred_element_type=jnp.float32)


def _reduce(x2):
    rows = x2.shape[0]
    grid = 10
    blk = rows // grid
    return pl.pallas_call(
        _reduce_body,
        grid=(grid,),
        in_specs=[pl.BlockSpec((blk, 128), lambda i: (i, 0))],
        out_specs=pl.BlockSpec((blk, 8), lambda i: (i, 0)),
        out_shape=jax.ShapeDtypeStruct((rows, 8), jnp.float32),
    )(x2)


def kernel(z, edge_index, node_id, W1, b1, W2, b2):
    e = edge_index.shape[1]
    rows = edge_index[0]
    cols = edge_index[1]
    xn = lax.dynamic_slice_in_dim(z, node_id, 1, axis=0)
    P, Q = _build_tables(z, W1, b1.reshape(1, -1), xn)
    w2b2 = jnp.concatenate([W2[:, 0], jnp.broadcast_to(b2 / L, (L,))])
    out16 = _sc_edge(P, Q, rows, cols, w2b2)
    out8 = _reduce(out16.reshape(e * L // 128, 128))
    return out8.reshape(e, 1)


# speedup vs baseline: 1.0120x; 1.0120x over previous
import functools

import jax
import jax.numpy as jnp
from jax import lax
from jax.experimental import pallas as pl
from jax.experimental.pallas import tpu as pltpu
from jax.experimental.pallas import tpu_sc as plsc

NC = 2
NS = 16
NW = NC * NS
L = 16


def _tables_body(z_ref, w1_ref, b1_ref, xn_ref, p_ref, q_ref):
    c = jnp.dot(xn_ref[...], w1_ref[256:384, :],
                preferred_element_type=jnp.float32) + b1_ref[...]
    c = c * 0.5
    p_ref[...] = jnp.dot(z_ref[...], w1_ref[0:128, :],
                         preferred_element_type=jnp.float32) + c
    q_ref[...] = jnp.dot(z_ref[...], w1_ref[128:256, :],
                         preferred_element_type=jnp.float32) + c


def _build_tables(z, W1, b1, xn):
    n, _ = z.shape
    h = W1.shape[1]
    out = jax.ShapeDtypeStruct((n, h), jnp.float32)
    return pl.pallas_call(_tables_body, out_shape=(out, out))(z, W1, b1, xn)


def _sc_edge_body(p_hbm, q_hbm, rows_hbm, cols_hbm, w2_hbm, out_hbm,
                  ridx0, ridx1, ridx2, ridx3, cidx0, cidx1, cidx2, cidx3,
                  qrows0, qrows1, prows0, prows1, out0, out1, w2_v,
                  ir0, ir1, ir2, ir3, ic0, ic1, ic2, ic3,
                  qs0, qs1, ps0, ps1, os0, os1,
                  *, edges_per_w, blk, hidden):
    wid = lax.axis_index("c") * NS + lax.axis_index("s")
    base = wid * edges_per_w
    nstep = edges_per_w // blk
    nchunk = hidden // L
    pltpu.sync_copy(w2_hbm, w2_v)

    ridx = (ridx0, ridx1, ridx2, ridx3)
    cidx = (cidx0, cidx1, cidx2, cidx3)
    irs = (ir0, ir1, ir2, ir3)
    ics = (ic0, ic1, ic2, ic3)
    qrows = (qrows0, qrows1)
    prows = (prows0, prows1)
    outs = (out0, out1)
    qsems = (qs0, qs1)
    psems = (ps0, ps1)
    osems = (os0, os1)

    def start_idx(j):
        s = j % 4
        off = base + j * blk
        dr = pltpu.make_async_copy(rows_hbm.at[pl.ds(off, blk)], ridx[s],
                                   irs[s])
        dc = pltpu.make_async_copy(cols_hbm.at[pl.ds(off, blk)], cidx[s],
                                   ics[s])
        dr.start()
        dc.start()
        return dr, dc

    def start_gathers(j):
        s, b = j % 4, j % 2
        dq = pltpu.make_async_copy(q_hbm.at[ridx[s]], qrows[b], qsems[b])
        dp = pltpu.make_async_copy(p_hbm.at[cidx[s]], prows[b], psems[b])
        dq.start()
        dp.start()
        return dq, dp

    def start_out(j):
        b = j % 2
        d = pltpu.make_async_copy(
            outs[b], out_hbm.at[pl.ds(base + j * blk, blk), :], osems[b])
        d.start()
        return d

    def compute(b):
        pr, qr, ob = prows[b], qrows[b], outs[b]
        b2v = w2_v[pl.ds(hidden, L)]

        @plsc.parallel_loop(0, blk, unroll=4)
        def _edge(e):
            s = b2v
            for ch in range(nchunk):
                sl = pl.ds(ch * L, L)
                s = s + jnp.maximum(pr[e, sl] + qr[e, sl], 0.0) * w2_v[sl]
            ob[e, :] = s

    i0 = start_idx(0)
    i1 = start_idx(1) if nstep > 1 else None
    i0[0].wait()
    i0[1].wait()
    g = start_gathers(0)
    g[0].wait()
    g[1].wait()
    out_pending = {}
    for j in range(nstep):
        g_next = None
        if j + 1 < nstep:
            if j == 0 and i1 is not None:
                i1[0].wait()
                i1[1].wait()
            g_next = start_gathers(j + 1)
        i_next = start_idx(j + 2) if j + 2 < nstep else None
        if j >= 2:
            out_pending.pop(j - 2).wait()
        compute(j % 2)
        out_pending[j] = start_out(j)
        if g_next is not None:
            g_next[0].wait()
            g_next[1].wait()
        if i_next is not None:
            i_next[0].wait()
            i_next[1].wait()
    for d in out_pending.values():
        d.wait()


def _sc_edge(P, Q, rows, cols, w2):
    n, hidden = P.shape
    e = rows.shape[0]
    edges_per_w = e // NW
    blk = 400
    mesh = plsc.VectorSubcoreMesh(core_axis_name="c", subcore_axis_name="s")
    body = functools.partial(_sc_edge_body, edges_per_w=edges_per_w, blk=blk,
                             hidden=hidden)
    idx_t = pltpu.VMEM((blk,), jnp.int32)
    tab_t = pltpu.VMEM((blk, hidden), jnp.float32)
    out_t = pltpu.VMEM((blk, L), jnp.float32)
    run = pl.kernel(
        body,
        out_type=jax.ShapeDtypeStruct((e, L), jnp.float32),
        mesh=mesh,
        compiler_params=pltpu.CompilerParams(use_tc_tiling_on_sc=False),
        scratch_types=(
            [idx_t] * 8 + [tab_t] * 4 + [out_t] * 2
            + [pltpu.VMEM((hidden + L,), jnp.float32)]
            + [pltpu.SemaphoreType.DMA] * 14
        ),
    )
    return run(P, Q, rows, cols, w2)


def _reduce_body(x_ref, o_ref):
    sel = (lax.broadcasted_iota(jnp.int32, (128, 8), 0) // L
           == lax.broadcasted_iota(jnp.int32, (128, 8), 1))
    o_ref[...] = jnp.dot(x_ref[...], sel.astype(jnp.float32),
                         preferred_element_type=jnp.float32)


def _reduce(x2):
    rows = x2.shape[0]
    grid = 10
    blk = rows // grid
    return pl.pallas_call(
        _reduce_body,
        grid=(grid,),
        in_specs=[pl.BlockSpec((blk, 128), lambda i: (i, 0))],
        out_specs=pl.BlockSpec((blk, 8), lambda i: (i, 0)),
        out_shape=jax.ShapeDtypeStruct((rows, 8), jnp.float32),
    )(x2)


def kernel(z, edge_index, node_id, W1, b1, W2, b2):
    e = edge_index.shape[1]
    rows = edge_index[0]
    cols = edge_index[1]
    xn = lax.dynamic_slice_in_dim(z, node_id, 1, axis=0)
    P, Q = _build_tables(z, W1, b1.reshape(1, -1), xn)
    w2b2 = jnp.concatenate([W2[:, 0], jnp.broadcast_to(b2 / L, (L,))])
    out16 = _sc_edge(P, Q, rows, cols, w2b2)
    out8 = _reduce(out16.reshape(e * L // 128, 128))
    return out8.reshape(e, 1)
